# Initial kernel scaffold; baseline (speedup 1.0000x reference)
import jax
import jax.numpy as jnp
from jax.experimental import pallas as pl


def kernel(x, edge_index, batch, W1, b1, W2, b2, Wo1, bo1, Wo2, bo2):
    raise NotImplementedError("write your pallas kernel here")


# speedup vs baseline: 12.9448x; 12.9448x over previous
import functools
import jax
import jax.numpy as jnp
from jax import lax
from jax.experimental import pallas as pl
from jax.experimental.pallas import tpu as pltpu
from jax.experimental.pallas import tpu_sc as plsc

N_NODES = 10000
N_EDGES = 320000
D = 128
N_GRAPHS = 64

NC = 2
NS = 16
NW = NC * NS

EDGES_PER_TILE = N_EDGES // NW
CHUNK = 80
NCHUNKS = EDGES_PER_TILE // CHUNK
ACC_ROWS = 10240
ROWS_PER_TILE = ACC_ROWS // NS

ROW_BLK = 1000
NBLK = N_NODES // ROW_BLK

@functools.lru_cache(maxsize=None)
def _sc_mesh():
  return plsc.VectorSubcoreMesh(
      core_axis_name="c", subcore_axis_name="s", num_cores=NC, num_subcores=NS)


def _deg_body(dst_hbm, ones_hbm, zeros_hbm, out_hbm, ones_v, dst_v, acc):
  c = lax.axis_index("c")
  s = lax.axis_index("s")
  wid = s * NC + c
  base = wid * EDGES_PER_TILE
  pltpu.sync_copy(zeros_hbm, acc.at[pl.ds(s * ROWS_PER_TILE, ROWS_PER_TILE)])
  pltpu.sync_copy(ones_hbm, ones_v)
  plsc.subcore_barrier()

  def step(i, carry):
    pltpu.sync_copy(dst_hbm.at[pl.ds(base + i * CHUNK, CHUNK)], dst_v)
    pltpu.sync_copy(ones_v, acc.at[dst_v], add=True)
    return carry

  lax.fori_loop(0, NCHUNKS, step, 0)
  plsc.subcore_barrier()
  pltpu.sync_copy(acc.at[pl.ds(s * ROWS_PER_TILE, ROWS_PER_TILE)],
                  out_hbm.at[c, pl.ds(s * ROWS_PER_TILE, ROWS_PER_TILE)])


@functools.lru_cache(maxsize=None)
def _deg_kernel():
  return pl.kernel(
      _deg_body,
      out_type=jax.ShapeDtypeStruct((NC, ACC_ROWS, 16), jnp.float32),
      mesh=_sc_mesh(),
      scratch_types=[
          pltpu.VMEM((CHUNK, 16), jnp.float32),
          pltpu.VMEM((CHUNK,), jnp.int32),
          pltpu.VMEM_SHARED((ACC_ROWS, 16), jnp.float32),
      ])


def _seg_body(y_hbm, src_hbm, dst_hbm, zeros_hbm, out_hbm,
              src_v, dst_v, rows_v, sem, acc):
  c = lax.axis_index("c")
  s = lax.axis_index("s")
  wid = s * NC + c
  base = wid * EDGES_PER_TILE
  pltpu.sync_copy(zeros_hbm, acc.at[pl.ds(s * ROWS_PER_TILE, ROWS_PER_TILE)])
  plsc.subcore_barrier()

  def step(i, carry):
    pltpu.sync_copy(src_hbm.at[pl.ds(base + i * CHUNK, CHUNK)], src_v)
    pltpu.sync_copy(dst_hbm.at[pl.ds(base + i * CHUNK, CHUNK)], dst_v)
    pltpu.async_copy(y_hbm.at[src_v], rows_v, sem).wait()
    pltpu.sync_copy(rows_v, acc.at[dst_v], add=True)
    return carry

  lax.fori_loop(0, NCHUNKS, step, 0)
  plsc.subcore_barrier()
  pltpu.sync_copy(acc.at[pl.ds(s * ROWS_PER_TILE, ROWS_PER_TILE)],
                  out_hbm.at[c, pl.ds(s * ROWS_PER_TILE, ROWS_PER_TILE)])


@functools.lru_cache(maxsize=None)
def _seg_kernel():
  return pl.kernel(
      _seg_body,
      out_type=jax.ShapeDtypeStruct((NC, ACC_ROWS, D), jnp.float32),
      mesh=_sc_mesh(),
      scratch_types=[
          pltpu.VMEM((CHUNK,), jnp.int32),
          pltpu.VMEM((CHUNK,), jnp.int32),
          pltpu.VMEM((CHUNK, D), jnp.float32),
          pltpu.SemaphoreType.DMA,
          pltpu.VMEM_SHARED((ACC_ROWS, D), jnp.float32),
      ])


def _dinv(deg_blk):
  return lax.rsqrt(deg_blk[0, :, 0:1] + deg_blk[1, :, 0:1] + 1.0)


def _tcA_body(x_ref, w_ref, deg_ref, y_ref):
  y_ref[...] = jnp.dot(x_ref[...], w_ref[...],
                       preferred_element_type=jnp.float32) * _dinv(deg_ref[...])


def _tcC_body(s_ref, y_ref, deg_ref, b_ref, w_ref, y2_ref):
  dinv = _dinv(deg_ref[...])
  h = s_ref[0] + s_ref[1] + y_ref[...]
  h = jnp.maximum(dinv * h + b_ref[...], 0.0)
  y2_ref[...] = jnp.dot(h, w_ref[...],
                        preferred_element_type=jnp.float32) * dinv


def _tcD_body(s_ref, y_ref, deg_ref, b_ref, batch_ref,
              wo1_ref, bo1_ref, wo2_ref, bo2_ref, out_ref, g_acc, cnt_acc):
  i = pl.program_id(0)

  @pl.when(i == 0)
  def _():
    g_acc[...] = jnp.zeros_like(g_acc)
    cnt_acc[...] = jnp.zeros_like(cnt_acc)

  dinv = _dinv(deg_ref[...])
  h = s_ref[0] + s_ref[1] + y_ref[...]
  h = jnp.maximum(dinv * h + b_ref[...], 0.0)
  b = batch_ref[0, 0, :]
  pt = (b[None, :] == lax.broadcasted_iota(jnp.int32, (N_GRAPHS, ROW_BLK), 0)
        ).astype(jnp.float32)
  g_acc[...] += lax.dot_general(pt, h, (((1,), (0,)), ((), ())),
                                preferred_element_type=jnp.float32)
  cnt_acc[...] += jnp.broadcast_to(jnp.sum(pt, axis=1)[:, None], (N_GRAPHS, D))

  @pl.when(i == NBLK - 1)
  def _():
    g = g_acc[...] / jnp.maximum(cnt_acc[...], 1.0)
    g = jnp.maximum(
        jnp.dot(g, wo1_ref[...], preferred_element_type=jnp.float32)
        + bo1_ref[...], 0.0)
    out_ref[...] = jnp.dot(g, wo2_ref[...],
                           preferred_element_type=jnp.float32) + bo2_ref[...]


_row_spec = pl.BlockSpec((ROW_BLK, D), lambda i: (i, 0))
_seg_spec = pl.BlockSpec((NC, ROW_BLK, D), lambda i: (0, i, 0))
_deg_spec = pl.BlockSpec((NC, ROW_BLK, 16), lambda i: (0, i, 0))
_w_spec = pl.BlockSpec((D, D), lambda i: (0, 0))
_b_spec = pl.BlockSpec((1, D), lambda i: (0, 0))

_tcA = pl.pallas_call(
    _tcA_body, grid=(NBLK,),
    in_specs=[_row_spec, _w_spec, _deg_spec],
    out_specs=_row_spec,
    out_shape=jax.ShapeDtypeStruct((N_NODES, D), jnp.float32))

_tcC = pl.pallas_call(
    _tcC_body, grid=(NBLK,),
    in_specs=[_seg_spec, _row_spec, _deg_spec, _b_spec, _w_spec],
    out_specs=_row_spec,
    out_shape=jax.ShapeDtypeStruct((N_NODES, D), jnp.float32))

_tcD = pl.pallas_call(
    _tcD_body, grid=(NBLK,),
    in_specs=[_seg_spec, _row_spec, _deg_spec, _b_spec,
              pl.BlockSpec((1, 1, ROW_BLK), lambda i: (i, 0, 0)),
              _w_spec, _b_spec, _w_spec, _b_spec],
    out_specs=pl.BlockSpec((N_GRAPHS, D), lambda i: (0, 0)),
    out_shape=jax.ShapeDtypeStruct((N_GRAPHS, D), jnp.float32),
    scratch_shapes=[pltpu.VMEM((N_GRAPHS, D), jnp.float32),
                    pltpu.VMEM((N_GRAPHS, D), jnp.float32)])


def kernel(x, edge_index, batch, W1, b1, W2, b2, Wo1, bo1, Wo2, bo2):
  src = edge_index[0].astype(jnp.int32)
  dst = edge_index[1].astype(jnp.int32)
  batch3 = batch.astype(jnp.int32).reshape(NBLK, 1, ROW_BLK)
  ones16 = jnp.ones((CHUNK, 16), jnp.float32)
  zeros16 = jnp.zeros((ROWS_PER_TILE, 16), jnp.float32)
  zeros128 = jnp.zeros((ROWS_PER_TILE, D), jnp.float32)
  b1r = b1.reshape(1, D)
  b2r = b2.reshape(1, D)
  bo1r = bo1.reshape(1, D)
  bo2r = bo2.reshape(1, D)

  deg = _deg_kernel()(dst, ones16, zeros16)
  y1 = _tcA(x, W1, deg)
  s1 = _seg_kernel()(y1, src, dst, zeros128)
  y2 = _tcC(s1, y1, deg, b1r, W2)
  s2 = _seg_kernel()(y2, src, dst, zeros128)
  return _tcD(s2, y2, deg, b2r, batch3, Wo1, bo1r, Wo2, bo2r)


# speedup vs baseline: 22.1271x; 1.7093x over previous
import functools
import jax
import jax.numpy as jnp
from jax import lax
from jax.experimental import pallas as pl
from jax.experimental.pallas import tpu as pltpu
from jax.experimental.pallas import tpu_sc as plsc

N_NODES = 10000
N_EDGES = 320000
D = 128
N_GRAPHS = 64

NC = 2
NS = 16
NW = NC * NS

EDGES_PER_TILE = N_EDGES // NW
CHUNK = 80
NCHUNKS = EDGES_PER_TILE // CHUNK
ACC_ROWS = 10240
ROWS_PER_TILE = ACC_ROWS // NS

ROW_BLK = 1000
NBLK = N_NODES // ROW_BLK

@functools.lru_cache(maxsize=None)
def _sc_mesh():
  return plsc.VectorSubcoreMesh(
      core_axis_name="c", subcore_axis_name="s", num_cores=NC, num_subcores=NS)


def _deg_body(dst_hbm, ones_hbm, zeros_hbm, out_hbm,
              ones_v, dst_v0, dst_v1, d_sem0, d_sem1, acc):
  c = lax.axis_index("c")
  s = lax.axis_index("s")
  wid = s * NC + c
  base = wid * EDGES_PER_TILE
  dst_v = (dst_v0, dst_v1)
  d_sem = (d_sem0, d_sem1)
  pltpu.sync_copy(ones_hbm, ones_v)
  pltpu.sync_copy(zeros_hbm, acc.at[pl.ds(s * ROWS_PER_TILE, ROWS_PER_TILE)])
  plsc.subcore_barrier()

  dst0 = pltpu.async_copy(dst_hbm.at[pl.ds(base, CHUNK)], dst_v[0], d_sem[0])
  dst0.wait()

  def step(j, carry):
    for b in (0, 1):
      cidx = 2 * j + b
      nxt = pltpu.async_copy(
          dst_hbm.at[pl.ds(base + (cidx + 1) * CHUNK, CHUNK)],
          dst_v[1 - b], d_sem[1 - b])
      pltpu.sync_copy(ones_v, acc.at[dst_v[b]], add=True)
      nxt.wait()
    return carry

  lax.fori_loop(0, NCHUNKS // 2, step, 0)
  pltpu.sync_copy(ones_v, acc.at[dst_v[0]], add=True)
  plsc.subcore_barrier()
  pltpu.sync_copy(acc.at[pl.ds(s * ROWS_PER_TILE, ROWS_PER_TILE)],
                  out_hbm.at[c, pl.ds(s * ROWS_PER_TILE, ROWS_PER_TILE)])


@functools.lru_cache(maxsize=None)
def _deg_kernel():
  return pl.kernel(
      _deg_body,
      out_type=jax.ShapeDtypeStruct((NC, ACC_ROWS, D), jnp.float32),
      mesh=_sc_mesh(),
      scratch_types=[
          pltpu.VMEM((CHUNK, D), jnp.float32),
          pltpu.VMEM((CHUNK,), jnp.int32),
          pltpu.VMEM((CHUNK,), jnp.int32),
          pltpu.SemaphoreType.DMA,
          pltpu.SemaphoreType.DMA,
          pltpu.VMEM_SHARED((ACC_ROWS, D), jnp.float32),
      ])


def _seg_body(y_hbm, src_hbm, dst_hbm, zeros_hbm, out_hbm,
              src_all, dst_v0, dst_v1, rows_v0, rows_v1,
              sp_sem, d_sem0, d_sem1, g_sem0, g_sem1, acc):
  c = lax.axis_index("c")
  s = lax.axis_index("s")
  wid = s * NC + c
  base = wid * EDGES_PER_TILE
  dst_v = (dst_v0, dst_v1)
  rows_v = (rows_v0, rows_v1)
  d_sem = (d_sem0, d_sem1)
  g_sem = (g_sem0, g_sem1)

  sp = pltpu.async_copy(src_hbm.at[wid], src_all, sp_sem)
  pltpu.sync_copy(zeros_hbm, acc.at[pl.ds(s * ROWS_PER_TILE, ROWS_PER_TILE)])
  sp.wait()
  plsc.subcore_barrier()

  def prefetch(cidx, b):
    d = pltpu.async_copy(dst_hbm.at[pl.ds(base + cidx * CHUNK, CHUNK)],
                         dst_v[b], d_sem[b])
    g = pltpu.async_copy(y_hbm.at[src_all.at[cidx]], rows_v[b], g_sem[b])
    return d, g

  d, g = prefetch(0, 0)
  d.wait()
  g.wait()

  def step(j, carry):
    for b in (0, 1):
      cidx = 2 * j + b
      d, g = prefetch(cidx + 1, 1 - b)
      pltpu.sync_copy(rows_v[b], acc.at[dst_v[b]], add=True)
      d.wait()
      g.wait()
    return carry

  lax.fori_loop(0, NCHUNKS // 2, step, 0)
  pltpu.sync_copy(rows_v[0], acc.at[dst_v[0]], add=True)
  plsc.subcore_barrier()
  pltpu.sync_copy(acc.at[pl.ds(s * ROWS_PER_TILE, ROWS_PER_TILE)],
                  out_hbm.at[c, pl.ds(s * ROWS_PER_TILE, ROWS_PER_TILE)])


@functools.lru_cache(maxsize=None)
def _seg_kernel():
  return pl.kernel(
      _seg_body,
      out_type=jax.ShapeDtypeStruct((NC, ACC_ROWS, D), jnp.float32),
      mesh=_sc_mesh(),
      scratch_types=[
          pltpu.VMEM((NCHUNKS, CHUNK), jnp.int32),
          pltpu.VMEM((CHUNK,), jnp.int32),
          pltpu.VMEM((CHUNK,), jnp.int32),
          pltpu.VMEM((CHUNK, D), jnp.float32),
          pltpu.VMEM((CHUNK, D), jnp.float32),
          pltpu.SemaphoreType.DMA,
          pltpu.SemaphoreType.DMA,
          pltpu.SemaphoreType.DMA,
          pltpu.SemaphoreType.DMA,
          pltpu.SemaphoreType.DMA,
          pltpu.VMEM_SHARED((ACC_ROWS, D), jnp.float32),
      ])


def _dinv(deg_blk):
  return lax.rsqrt(deg_blk[0] + deg_blk[1] + 1.0)


def _tcA_body(x_ref, w_ref, deg_ref, y_ref):
  y_ref[...] = jnp.dot(x_ref[...], w_ref[...],
                       preferred_element_type=jnp.float32) * _dinv(deg_ref[...])


def _tcC_body(s_ref, y_ref, deg_ref, b_ref, w_ref, y2_ref):
  dinv = _dinv(deg_ref[...])
  h = s_ref[0] + s_ref[1] + y_ref[...]
  h = jnp.maximum(dinv * h + b_ref[...], 0.0)
  y2_ref[...] = jnp.dot(h, w_ref[...],
                        preferred_element_type=jnp.float32) * dinv


def _tcD_body(s_ref, y_ref, deg_ref, b_ref, batch_ref,
              wo1_ref, bo1_ref, wo2_ref, bo2_ref, out_ref, g_acc, cnt_acc):
  i = pl.program_id(0)

  @pl.when(i == 0)
  def _():
    g_acc[...] = jnp.zeros_like(g_acc)
    cnt_acc[...] = jnp.zeros_like(cnt_acc)

  dinv = _dinv(deg_ref[...])
  h = s_ref[0] + s_ref[1] + y_ref[...]
  h = jnp.maximum(dinv * h + b_ref[...], 0.0)
  b = batch_ref[0, 0, :]
  pt = (b[None, :] == lax.broadcasted_iota(jnp.int32, (N_GRAPHS, ROW_BLK), 0)
        ).astype(jnp.float32)
  g_acc[...] += lax.dot_general(pt, h, (((1,), (0,)), ((), ())),
                                preferred_element_type=jnp.float32)
  cnt_acc[...] += jnp.broadcast_to(jnp.sum(pt, axis=1)[:, None], (N_GRAPHS, D))

  @pl.when(i == NBLK - 1)
  def _():
    g = g_acc[...] / jnp.maximum(cnt_acc[...], 1.0)
    g = jnp.maximum(
        jnp.dot(g, wo1_ref[...], preferred_element_type=jnp.float32)
        + bo1_ref[...], 0.0)
    out_ref[...] = jnp.dot(g, wo2_ref[...],
                           preferred_element_type=jnp.float32) + bo2_ref[...]


_row_spec = pl.BlockSpec((ROW_BLK, D), lambda i: (i, 0))
_seg_spec = pl.BlockSpec((NC, ROW_BLK, D), lambda i: (0, i, 0))
_deg_spec = pl.BlockSpec((NC, ROW_BLK, D), lambda i: (0, i, 0))
_w_spec = pl.BlockSpec((D, D), lambda i: (0, 0))
_b_spec = pl.BlockSpec((1, D), lambda i: (0, 0))

_tcA = pl.pallas_call(
    _tcA_body, grid=(NBLK,),
    in_specs=[_row_spec, _w_spec, _deg_spec],
    out_specs=_row_spec,
    out_shape=jax.ShapeDtypeStruct((N_NODES, D), jnp.float32))

_tcC = pl.pallas_call(
    _tcC_body, grid=(NBLK,),
    in_specs=[_seg_spec, _row_spec, _deg_spec, _b_spec, _w_spec],
    out_specs=_row_spec,
    out_shape=jax.ShapeDtypeStruct((N_NODES, D), jnp.float32))

_tcD = pl.pallas_call(
    _tcD_body, grid=(NBLK,),
    in_specs=[_seg_spec, _row_spec, _deg_spec, _b_spec,
              pl.BlockSpec((1, 1, ROW_BLK), lambda i: (i, 0, 0)),
              _w_spec, _b_spec, _w_spec, _b_spec],
    out_specs=pl.BlockSpec((N_GRAPHS, D), lambda i: (0, 0)),
    out_shape=jax.ShapeDtypeStruct((N_GRAPHS, D), jnp.float32),
    scratch_shapes=[pltpu.VMEM((N_GRAPHS, D), jnp.float32),
                    pltpu.VMEM((N_GRAPHS, D), jnp.float32)])


def kernel(x, edge_index, batch, W1, b1, W2, b2, Wo1, bo1, Wo2, bo2):
  src = edge_index[0].astype(jnp.int32)
  dst = edge_index[1].astype(jnp.int32)
  batch3 = batch.astype(jnp.int32).reshape(NBLK, 1, ROW_BLK)
  zeros128 = jnp.zeros((ROWS_PER_TILE, D), jnp.float32)
  ones128 = jnp.ones((CHUNK, D), jnp.float32)
  b1r = b1.reshape(1, D)
  b2r = b2.reshape(1, D)
  bo1r = bo1.reshape(1, D)
  bo2r = bo2.reshape(1, D)

  src3 = src.reshape(NW, NCHUNKS, CHUNK)
  deg = _deg_kernel()(dst, ones128, zeros128)
  y1 = _tcA(x, W1, deg)
  s1 = _seg_kernel()(y1, src3, dst, zeros128)
  y2 = _tcC(s1, y1, deg, b1r, W2)
  s2 = _seg_kernel()(y2, src3, dst, zeros128)
  return _tcD(s2, y2, deg, b2r, batch3, Wo1, bo1r, Wo2, bo2r)
